# Initial kernel scaffold; baseline (speedup 1.0000x reference)
import jax
import jax.numpy as jnp
from jax.experimental import pallas as pl


def kernel(preds, targets):
    raise NotImplementedError("write your pallas kernel here")


# speedup vs baseline: 7.2180x; 7.2180x over previous
import functools

import jax
import jax.numpy as jnp
from jax import lax
from jax.experimental import pallas as pl
from jax.experimental.pallas import tpu as pltpu
from jax.experimental.pallas import tpu_sc as plsc

N_BATCH = 4
N_CLASSES = 19
SP = 512 * 512
N_PIX = N_BATCH * SP
TOPK = int(0.8 * N_PIX)

NW = 32
CHUNK = N_PIX // NW
LANES = 16
VECS = CHUNK // LANES
NBINS = 32768
BIN_VECS = NBINS // LANES

W_CE = 16384


def _ce_body(p_ref, t_ref, o_ref):
    p = p_ref[...]
    t = t_ref[...]
    m = jnp.max(p, axis=1, keepdims=True)
    s = jnp.sum(jnp.exp(p - m), axis=1, keepdims=True)
    cls = lax.broadcasted_iota(jnp.int32, p.shape, 1)
    pt = jnp.sum(jnp.where(cls == t, p, 0.0), axis=1, keepdims=True)
    o_ref[...] = m + jnp.log(s) - pt


_ce = pl.pallas_call(
    _ce_body,
    grid=(N_BATCH, SP // W_CE),
    in_specs=[
        pl.BlockSpec((1, N_CLASSES, W_CE), lambda b, j: (b, 0, j)),
        pl.BlockSpec((1, 1, W_CE), lambda b, j: (b, 0, j)),
    ],
    out_specs=pl.BlockSpec((1, 1, W_CE), lambda b, j: (b, 0, j)),
    out_shape=jax.ShapeDtypeStruct((N_BATCH, 1, SP), jnp.float32),
)


def _sc_hist_body(loss_hbm, cnt_hbm, sum_hbm, data_v, cnt_v, sum_v):
    wid = lax.axis_index("s") * 2 + lax.axis_index("c")
    pltpu.sync_copy(loss_hbm.at[pl.ds(wid * CHUNK, CHUNK)], data_v)

    zeros = jnp.zeros((LANES,), jnp.float32)

    def zero_body(i, carry):
        cnt_v[pl.ds(i * LANES, LANES)] = zeros
        sum_v[pl.ds(i * LANES, LANES)] = zeros
        return carry

    lax.fori_loop(0, BIN_VECS, zero_body, 0)

    ones = jnp.full((LANES,), 1.0, jnp.float32)
    izero = jnp.zeros((LANES,), jnp.int32)

    def hist_body(i, carry):
        v = data_v[pl.ds(i * LANES, LANES)]
        bits = lax.bitcast_convert_type(v, jnp.int32)
        bins = lax.shift_right_logical(lax.max(bits, izero), 16)
        plsc.addupdate_scatter(cnt_v, [bins], ones)
        plsc.addupdate_scatter(sum_v, [bins], v)
        return carry

    lax.fori_loop(0, VECS, hist_body, 0)

    pltpu.sync_copy(cnt_v, cnt_hbm.at[pl.ds(wid * NBINS, NBINS)])
    pltpu.sync_copy(sum_v, sum_hbm.at[pl.ds(wid * NBINS, NBINS)])


@functools.cache
def _sc_hist():
    return pl.kernel(
        _sc_hist_body,
        mesh=plsc.VectorSubcoreMesh(core_axis_name="c", subcore_axis_name="s"),
        out_type=[
            jax.ShapeDtypeStruct((NW * NBINS,), jnp.float32),
            jax.ShapeDtypeStruct((NW * NBINS,), jnp.float32),
        ],
        scratch_types=[
            pltpu.VMEM((CHUNK,), jnp.float32),
            pltpu.VMEM((NBINS,), jnp.float32),
            pltpu.VMEM((NBINS,), jnp.float32),
        ],
        compiler_params=pltpu.CompilerParams(needs_layout_passes=False),
    )


def _cumsum_shift(x, axis):
    n = x.shape[axis]
    sh = 1
    while sh < n:
        zeros = lax.slice_in_dim(jnp.zeros_like(x), 0, sh, axis=axis)
        shifted = lax.slice_in_dim(x, 0, n - sh, axis=axis)
        x = x + lax.concatenate([zeros, shifted], dimension=axis)
        sh *= 2
    return x


def _sel_body(c_ref, s_ref, o_ref):
    c = jnp.sum(c_ref[...], axis=0)
    s = jnp.sum(s_ref[...], axis=0)
    ce0 = _cumsum_shift(c, 0) - c
    row_off = jnp.sum(ce0, axis=1, keepdims=True)
    pe = (_cumsum_shift(c, 1) - c) + row_off
    total = jnp.sum(c)
    suf = total - pe
    r = lax.broadcasted_iota(jnp.int32, c.shape, 0)
    l = lax.broadcasted_iota(jnp.int32, c.shape, 1)
    bidx = r * 128 + l
    kf = jnp.float32(TOPK)
    bsel = jnp.max(jnp.where(suf >= kf, bidx, -1))
    above = bidx > bsel
    at = bidx == bsel
    c_above = jnp.sum(jnp.where(above, c, 0.0))
    s_above = jnp.sum(jnp.where(above, s, 0.0))
    c_bin = jnp.sum(jnp.where(at, c, 0.0))
    s_bin = jnp.sum(jnp.where(at, s, 0.0))
    m = kf - c_above
    res = (s_above + m * (s_bin / jnp.maximum(c_bin, 1.0))) / kf
    o_ref[...] = jnp.broadcast_to(res, (1, 1))


_sel = pl.pallas_call(
    _sel_body,
    in_specs=[
        pl.BlockSpec((NW, NBINS // 128, 128), lambda: (0, 0, 0)),
        pl.BlockSpec((NW, NBINS // 128, 128), lambda: (0, 0, 0)),
    ],
    out_specs=pl.BlockSpec((1, 1), lambda: (0, 0)),
    out_shape=jax.ShapeDtypeStruct((1, 1), jnp.float32),
)


def kernel(preds, targets):
    p = preds.reshape(N_BATCH, N_CLASSES, SP)
    t = targets.astype(jnp.int32).reshape(N_BATCH, 1, SP)
    loss = _ce(p, t).reshape(N_PIX)
    cnt, sm = _sc_hist()(loss)
    out = _sel(
        cnt.reshape(NW, NBINS // 128, 128), sm.reshape(NW, NBINS // 128, 128)
    )
    return out[0, 0]


# speedup vs baseline: 19.5400x; 2.7071x over previous
import functools

import jax
import jax.numpy as jnp
from jax import lax
from jax.experimental import pallas as pl
from jax.experimental.pallas import tpu as pltpu
from jax.experimental.pallas import tpu_sc as plsc

N_BATCH = 4
N_CLASSES = 19
SP = 512 * 512
N_PIX = N_BATCH * SP
TOPK = int(0.8 * N_PIX)

NW = 32
CHUNK = N_PIX // NW
LANES = 16
VECS = CHUNK // LANES
NBINS = 32768
BIN_VECS = NBINS // LANES

W_CE = 16384


R_CE = 256


def _ce_body(p_ref, t_ref, o_ref):
    p = p_ref[...]
    t = t_ref[...][:, None, :, :]
    m = jnp.max(p, axis=1, keepdims=True)
    s = jnp.sum(jnp.exp(p - m), axis=1, keepdims=True)
    cls = lax.broadcasted_iota(jnp.int32, p.shape, 1)
    pt = jnp.sum(jnp.where(cls == t, p, 0.0), axis=1, keepdims=True)
    o_ref[...] = (m + jnp.log(s) - pt)[:, 0, :, :]


_ce = pl.pallas_call(
    _ce_body,
    grid=(N_BATCH, 512 // R_CE),
    in_specs=[
        pl.BlockSpec((1, N_CLASSES, R_CE, 512), lambda b, j: (b, 0, j, 0)),
        pl.BlockSpec((1, R_CE, 512), lambda b, j: (b, j, 0)),
    ],
    out_specs=pl.BlockSpec((1, R_CE, 512), lambda b, j: (b, j, 0)),
    out_shape=jax.ShapeDtypeStruct((N_BATCH, 512, 512), jnp.float32),
    compiler_params=pltpu.CompilerParams(
        dimension_semantics=("parallel", "parallel")
    ),
)


def _sc_hist_body(loss_hbm, cnt_hbm, sum_hbm, data_v, cnt_v, sum_v):
    wid = lax.axis_index("s") * 2 + lax.axis_index("c")
    pltpu.sync_copy(loss_hbm.at[pl.ds(wid * CHUNK, CHUNK)], data_v)

    zeros = jnp.zeros((LANES,), jnp.float32)

    def zero_body(i, carry):
        cnt_v[pl.ds(i * LANES, LANES)] = zeros
        sum_v[pl.ds(i * LANES, LANES)] = zeros
        return carry

    lax.fori_loop(0, BIN_VECS, zero_body, 0)

    ones = jnp.full((LANES,), 1.0, jnp.float32)
    izero = jnp.zeros((LANES,), jnp.int32)

    def hist_body(i, carry):
        v = data_v[pl.ds(i * LANES, LANES)]
        bits = lax.bitcast_convert_type(v, jnp.int32)
        bins = lax.shift_right_logical(lax.max(bits, izero), 16)
        plsc.addupdate_scatter(cnt_v, [bins], ones)
        plsc.addupdate_scatter(sum_v, [bins], v)
        return carry

    lax.fori_loop(0, VECS, hist_body, 0)

    pltpu.sync_copy(cnt_v, cnt_hbm.at[pl.ds(wid * NBINS, NBINS)])
    pltpu.sync_copy(sum_v, sum_hbm.at[pl.ds(wid * NBINS, NBINS)])


@functools.cache
def _sc_hist():
    return pl.kernel(
        _sc_hist_body,
        mesh=plsc.VectorSubcoreMesh(core_axis_name="c", subcore_axis_name="s"),
        out_type=[
            jax.ShapeDtypeStruct((NW * NBINS,), jnp.float32),
            jax.ShapeDtypeStruct((NW * NBINS,), jnp.float32),
        ],
        scratch_types=[
            pltpu.VMEM((CHUNK,), jnp.float32),
            pltpu.VMEM((NBINS,), jnp.float32),
            pltpu.VMEM((NBINS,), jnp.float32),
        ],
        compiler_params=pltpu.CompilerParams(needs_layout_passes=False),
    )


def _cumsum_shift(x, axis):
    n = x.shape[axis]
    sh = 1
    while sh < n:
        zeros = lax.slice_in_dim(jnp.zeros_like(x), 0, sh, axis=axis)
        shifted = lax.slice_in_dim(x, 0, n - sh, axis=axis)
        x = x + lax.concatenate([zeros, shifted], dimension=axis)
        sh *= 2
    return x


def _sel_body(c_ref, s_ref, o_ref):
    c = jnp.sum(c_ref[...], axis=0)
    s = jnp.sum(s_ref[...], axis=0)
    ce0 = _cumsum_shift(c, 0) - c
    row_off = jnp.sum(ce0, axis=1, keepdims=True)
    pe = (_cumsum_shift(c, 1) - c) + row_off
    total = jnp.sum(c)
    suf = total - pe
    r = lax.broadcasted_iota(jnp.int32, c.shape, 0)
    l = lax.broadcasted_iota(jnp.int32, c.shape, 1)
    bidx = r * 128 + l
    kf = jnp.float32(TOPK)
    bsel = jnp.max(jnp.where(suf >= kf, bidx, -1))
    above = bidx > bsel
    at = bidx == bsel
    c_above = jnp.sum(jnp.where(above, c, 0.0))
    s_above = jnp.sum(jnp.where(above, s, 0.0))
    c_bin = jnp.sum(jnp.where(at, c, 0.0))
    s_bin = jnp.sum(jnp.where(at, s, 0.0))
    m = kf - c_above
    res = (s_above + m * (s_bin / jnp.maximum(c_bin, 1.0))) / kf
    o_ref[...] = jnp.broadcast_to(res, (1, 1))


_sel = pl.pallas_call(
    _sel_body,
    in_specs=[
        pl.BlockSpec((NW, NBINS // 128, 128), lambda: (0, 0, 0)),
        pl.BlockSpec((NW, NBINS // 128, 128), lambda: (0, 0, 0)),
    ],
    out_specs=pl.BlockSpec((1, 1), lambda: (0, 0)),
    out_shape=jax.ShapeDtypeStruct((1, 1), jnp.float32),
)


def kernel(preds, targets):
    t = targets.astype(jnp.int32)
    loss = _ce(preds, t).reshape(N_PIX)
    cnt, sm = _sc_hist()(loss)
    out = _sel(
        cnt.reshape(NW, NBINS // 128, 128), sm.reshape(NW, NBINS // 128, 128)
    )
    return out[0, 0]


# speedup vs baseline: 20.6121x; 1.0549x over previous
import functools

import jax
import jax.numpy as jnp
from jax import lax
from jax.experimental import pallas as pl
from jax.experimental.pallas import tpu as pltpu
from jax.experimental.pallas import tpu_sc as plsc

N_BATCH = 4
N_CLASSES = 19
SP = 512 * 512
N_PIX = N_BATCH * SP
TOPK = int(0.8 * N_PIX)

NW = 32
CHUNK = N_PIX // NW
LANES = 16
VECS = CHUNK // LANES
NBINS = 32768
BIN_VECS = NBINS // LANES

W_CE = 16384


R_CE = 256


def _ce_body(p_ref, t_ref, o_ref):
    p = p_ref[...]
    t = t_ref[...][:, None, :, :]
    m = jnp.max(p, axis=1, keepdims=True)
    s = jnp.sum(jnp.exp(p - m), axis=1, keepdims=True)
    cls = lax.broadcasted_iota(jnp.int32, p.shape, 1)
    pt = jnp.sum(jnp.where(cls == t, p, 0.0), axis=1, keepdims=True)
    o_ref[...] = (m + jnp.log(s) - pt)[:, 0, :, :]


_ce = pl.pallas_call(
    _ce_body,
    grid=(N_BATCH, 512 // R_CE),
    in_specs=[
        pl.BlockSpec((1, N_CLASSES, R_CE, 512), lambda b, j: (b, 0, j, 0)),
        pl.BlockSpec((1, R_CE, 512), lambda b, j: (b, j, 0)),
    ],
    out_specs=pl.BlockSpec((1, R_CE, 512), lambda b, j: (b, j, 0)),
    out_shape=jax.ShapeDtypeStruct((N_BATCH, 512, 512), jnp.float32),
    compiler_params=pltpu.CompilerParams(
        dimension_semantics=("parallel", "parallel")
    ),
)


def _sc_hist_body(loss_hbm, cnt_hbm, sum_hbm, data_v, cnt_v, sum_v):
    wid = lax.axis_index("s") * 2 + lax.axis_index("c")
    pltpu.sync_copy(loss_hbm.at[pl.ds(wid * CHUNK, CHUNK)], data_v)

    zeros = jnp.zeros((LANES,), jnp.float32)
    zunroll = 16

    def zero_body(i, carry):
        for u in range(zunroll):
            off = (i * zunroll + u) * LANES
            cnt_v[pl.ds(off, LANES)] = zeros
            sum_v[pl.ds(off, LANES)] = zeros
        return carry

    lax.fori_loop(0, BIN_VECS // zunroll, zero_body, 0)

    ones = jnp.full((LANES,), 1.0, jnp.float32)
    izero = jnp.zeros((LANES,), jnp.int32)
    hunroll = 8

    def hist_body(i, carry):
        for u in range(hunroll):
            off = (i * hunroll + u) * LANES
            v = data_v[pl.ds(off, LANES)]
            bits = lax.bitcast_convert_type(v, jnp.int32)
            bins = lax.shift_right_logical(lax.max(bits, izero), 16)
            plsc.addupdate_scatter(cnt_v, [bins], ones)
            plsc.addupdate_scatter(sum_v, [bins], v)
        return carry

    lax.fori_loop(0, VECS // hunroll, hist_body, 0)

    pltpu.sync_copy(cnt_v, cnt_hbm.at[pl.ds(wid * NBINS, NBINS)])
    pltpu.sync_copy(sum_v, sum_hbm.at[pl.ds(wid * NBINS, NBINS)])


@functools.cache
def _sc_hist():
    return pl.kernel(
        _sc_hist_body,
        mesh=plsc.VectorSubcoreMesh(core_axis_name="c", subcore_axis_name="s"),
        out_type=[
            jax.ShapeDtypeStruct((NW * NBINS,), jnp.float32),
            jax.ShapeDtypeStruct((NW * NBINS,), jnp.float32),
        ],
        scratch_types=[
            pltpu.VMEM((CHUNK,), jnp.float32),
            pltpu.VMEM((NBINS,), jnp.float32),
            pltpu.VMEM((NBINS,), jnp.float32),
        ],
        compiler_params=pltpu.CompilerParams(needs_layout_passes=False),
    )


def _cumsum_shift(x, axis):
    n = x.shape[axis]
    sh = 1
    while sh < n:
        zeros = lax.slice_in_dim(jnp.zeros_like(x), 0, sh, axis=axis)
        shifted = lax.slice_in_dim(x, 0, n - sh, axis=axis)
        x = x + lax.concatenate([zeros, shifted], dimension=axis)
        sh *= 2
    return x


def _sel_body(c_ref, s_ref, o_ref):
    c = jnp.sum(c_ref[...], axis=0)
    s = jnp.sum(s_ref[...], axis=0)
    ce0 = _cumsum_shift(c, 0) - c
    row_off = jnp.sum(ce0, axis=1, keepdims=True)
    pe = (_cumsum_shift(c, 1) - c) + row_off
    total = jnp.sum(c)
    suf = total - pe
    r = lax.broadcasted_iota(jnp.int32, c.shape, 0)
    l = lax.broadcasted_iota(jnp.int32, c.shape, 1)
    bidx = r * 128 + l
    kf = jnp.float32(TOPK)
    bsel = jnp.max(jnp.where(suf >= kf, bidx, -1))
    above = bidx > bsel
    at = bidx == bsel
    c_above = jnp.sum(jnp.where(above, c, 0.0))
    s_above = jnp.sum(jnp.where(above, s, 0.0))
    c_bin = jnp.sum(jnp.where(at, c, 0.0))
    s_bin = jnp.sum(jnp.where(at, s, 0.0))
    m = kf - c_above
    res = (s_above + m * (s_bin / jnp.maximum(c_bin, 1.0))) / kf
    o_ref[...] = jnp.broadcast_to(res, (1, 1))


_sel = pl.pallas_call(
    _sel_body,
    in_specs=[
        pl.BlockSpec((NW, NBINS // 128, 128), lambda: (0, 0, 0)),
        pl.BlockSpec((NW, NBINS // 128, 128), lambda: (0, 0, 0)),
    ],
    out_specs=pl.BlockSpec((1, 1), lambda: (0, 0)),
    out_shape=jax.ShapeDtypeStruct((1, 1), jnp.float32),
)


def kernel(preds, targets):
    t = targets.astype(jnp.int32)
    loss = _ce(preds, t).reshape(N_PIX)
    cnt, sm = _sc_hist()(loss)
    out = _sel(
        cnt.reshape(NW, NBINS // 128, 128), sm.reshape(NW, NBINS // 128, 128)
    )
    return out[0, 0]
